# baseline (speedup 1.0000x reference)
import functools

import jax
import jax.numpy as jnp
from jax import lax
from jax.experimental import pallas as pl
from jax.experimental.pallas import tpu as pltpu
from jax.experimental.pallas import tpu_sc as plsc

HID = 64
PACK = 80
RN = 5000
BE = 8000
NC, NS = 2, 16
PREC = jax.lax.Precision.DEFAULT


def _silu(v):
    return v * jax.nn.sigmoid(v)


def _prep(p, ein, we=None, be=None):
    W0 = p['edge_mlp0']['w']
    b0 = p['edge_mlp0']['b']
    W_ea = W0[:, 2 * HID + 1:]
    if ein == 1:
        u = W_ea[:, 0]
        v = jnp.zeros((HID,), jnp.float32)
    else:
        u = W_ea @ we
        v = W_ea @ be
    return {
        'WhrT': W0[:, :HID].T, 'WhcT': W0[:, HID:2 * HID].T,
        'c1': (b0 + v)[None], 'wrad': W0[:, 2 * HID][None], 'u': u[None],
        'W1T': p['edge_mlp1']['w'].T, 'b1': p['edge_mlp1']['b'][None],
        'Wc0T': p['coord_mlp0']['w'].T, 'bc0': p['coord_mlp0']['b'][None],
        'wc1T': p['coord_mlp1']['w'].T,
        'Wn0aT': p['node_mlp0']['w'][:, :HID].T,
        'Wn0bT': p['node_mlp0']['w'][:, HID:].T,
        'bn0': p['node_mlp0']['b'][None],
        'Wn1T': p['node_mlp1']['w'].T, 'bn1': p['node_mlp1']['b'][None],
    }


def _lin_call(x, w, b):
    n, din = x.shape
    h = w.shape[0]
    def body(x_ref, wt_ref, b_ref, o_ref):
        o_ref[...] = jnp.dot(x_ref[...], wt_ref[...], precision=PREC,
                             preferred_element_type=jnp.float32) + b_ref[...]
    return pl.pallas_call(
        body, grid=(n // RN,),
        in_specs=[pl.BlockSpec((RN, din), lambda i: (i, 0)),
                  pl.BlockSpec((din, h), lambda i: (0, 0)),
                  pl.BlockSpec((1, h), lambda i: (0, 0))],
        out_specs=pl.BlockSpec((RN, h), lambda i: (i, 0)),
        out_shape=jax.ShapeDtypeStruct((n, h), jnp.float32),
    )(x, w.T, b[None])


def _node_call(hin, coord, upd=None, film=None, proj=None,
               emit_mean=False, emit_coord=True):
    n = hin.shape[0]
    grid = n // RN
    row_map = lambda i: (i, 0)
    const_map = lambda i: (0, 0)
    ops, specs = [], []

    def add(a, blk):
        ops.append(a)
        specs.append(pl.BlockSpec(blk, const_map if blk[0] != RN else row_map))

    add(hin, (RN, HID))
    add(coord, (RN, 3))
    if upd is not None:
        agg, w = upd
        add(agg, (RN, PACK))
        add(w['Wn0aT'], (HID, HID)); add(w['Wn0bT'], (HID, HID))
        add(w['bn0'], (1, HID))
        add(w['Wn1T'], (HID, HID)); add(w['bn1'], (1, HID))
    if film is not None:
        add(film[0], (1, HID)); add(film[1], (1, HID))
    if proj is not None:
        add(proj['WhrT'], (HID, HID)); add(proj['WhcT'], (HID, HID))
        add(proj['c1'], (1, HID))

    out_shapes = [jax.ShapeDtypeStruct((n, HID), jnp.float32)]
    out_specs = [pl.BlockSpec((RN, HID), row_map)]
    if upd is not None and emit_coord:
        out_shapes.append(jax.ShapeDtypeStruct((n, 3), jnp.float32))
        out_specs.append(pl.BlockSpec((RN, 3), row_map))
    if proj is not None:
        for _ in range(2):
            out_shapes.append(jax.ShapeDtypeStruct((n, PACK), jnp.float32))
            out_specs.append(pl.BlockSpec((RN, PACK), row_map))
    if emit_mean:
        out_shapes.append(jax.ShapeDtypeStruct((1, HID), jnp.float32))
        out_specs.append(pl.BlockSpec((1, HID), const_map))

    def body(*refs):
        it = iter(refs)
        h_ref = next(it); c_ref = next(it)
        hv = h_ref[...]
        cv = c_ref[...]
        if upd is not None:
            av = next(it)[...]
            wn0a = next(it)[...]; wn0b = next(it)[...]; bn0 = next(it)[...]
            wn1 = next(it)[...]; bn1 = next(it)[...]
            m = _silu(jnp.dot(hv, wn0a, precision=PREC,
                              preferred_element_type=jnp.float32)
                      + jnp.dot(av[:, :HID], wn0b, precision=PREC,
                                preferred_element_type=jnp.float32) + bn0)
            hv = hv + jnp.dot(m, wn1, precision=PREC,
                              preferred_element_type=jnp.float32) + bn1
            cv = cv + av[:, HID:HID + 3] / jnp.maximum(av[:, HID + 3:HID + 4], 1.0)
        if film is not None:
            sc = next(it)[...]; bb = next(it)[...]
            hv = sc * hv + bb
        if proj is not None:
            whr = next(it)[...]; whc = next(it)[...]; c1 = next(it)[...]
        hf_ref = next(it)
        hf_ref[...] = hv
        if upd is not None and emit_coord:
            next(it)[...] = cv
        if proj is not None:
            t1 = jnp.dot(hv, whr, precision=PREC,
                         preferred_element_type=jnp.float32) + c1
            t2 = jnp.dot(hv, whc, precision=PREC,
                         preferred_element_type=jnp.float32)
            z = jnp.zeros((RN, PACK - HID - 3), jnp.float32)
            next(it)[...] = jnp.concatenate([t1, cv, z], axis=1)
            next(it)[...] = jnp.concatenate([t2, -cv, z], axis=1)
        if emit_mean:
            ms_ref = next(it)
            @pl.when(pl.program_id(0) == 0)
            def _():
                ms_ref[...] = jnp.zeros_like(ms_ref)
            ms_ref[...] += jnp.sum(hv, axis=0, keepdims=True)

    out = pl.pallas_call(
        body, grid=(grid,), in_specs=specs, out_specs=out_specs,
        out_shape=out_shapes,
    )(*ops)
    return out


def _edge_call(g1, g2, ea2, pp, normalize):
    e = g1.shape[0]

    def body(g1_ref, g2_ref, ea_ref, wrad, u, w1, b1, wc0, bc0, wc1, o_ref):
        a = g1_ref[...]
        b = g2_ref[...]
        hsum = a[:, :HID] + b[:, :HID]
        diff = a[:, HID:HID + 3] + b[:, HID:HID + 3]
        radial = jnp.sum(diff * diff, axis=1, keepdims=True)
        eav = ea_ref[...]
        y0 = hsum + radial * wrad[...] + eav * u[...]
        ef = _silu(y0)
        edge_feat = _silu(jnp.dot(ef, w1[...], precision=PREC,
                                  preferred_element_type=jnp.float32) + b1[...])
        t = _silu(jnp.dot(edge_feat, wc0[...], precision=PREC,
                          preferred_element_type=jnp.float32) + bc0[...])
        s = jnp.dot(t, wc1[...], precision=PREC,
                    preferred_element_type=jnp.float32)
        if normalize:
            diff = diff / (jnp.sqrt(radial) + 1e-8)
        trans = diff * s
        o_ref[...] = jnp.concatenate(
            [edge_feat, trans, jnp.ones((BE, 1), jnp.float32),
             jnp.zeros((BE, PACK - HID - 4), jnp.float32)], axis=1)

    row_map = lambda i: (i, 0)
    const_map = lambda i: (0, 0)
    return pl.pallas_call(
        body, grid=(e // BE,),
        in_specs=[pl.BlockSpec((BE, PACK), row_map),
                  pl.BlockSpec((BE, PACK), row_map),
                  pl.BlockSpec((BE, 1), row_map),
                  pl.BlockSpec((1, HID), const_map),
                  pl.BlockSpec((1, HID), const_map),
                  pl.BlockSpec((HID, HID), const_map),
                  pl.BlockSpec((1, HID), const_map),
                  pl.BlockSpec((HID, HID), const_map),
                  pl.BlockSpec((1, HID), const_map),
                  pl.BlockSpec((HID, 1), const_map)],
        out_specs=pl.BlockSpec((BE, PACK), row_map),
        out_shape=jax.ShapeDtypeStruct((e, PACK), jnp.float32),
    )(g1, g2, ea2, pp['wrad'], pp['u'], pp['W1T'], pp['b1'],
      pp['Wc0T'], pp['bc0'], pp['wc1T'])


def _sc_gather(t1, t2, row, col):
    e = row.shape[0]
    nw = NC * NS
    per = e // nw
    SB = 512
    nb = (per + SB - 1) // SB
    mesh = plsc.VectorSubcoreMesh(core_axis_name="c", subcore_axis_name="s",
                                  num_cores=NC, num_subcores=NS)

    @functools.partial(
        pl.kernel, mesh=mesh,
        compiler_params=pltpu.CompilerParams(use_tc_tiling_on_sc=False),
        out_type=[jax.ShapeDtypeStruct((e, PACK), jnp.float32),
                  jax.ShapeDtypeStruct((e, PACK), jnp.float32)],
        scratch_types=[pltpu.VMEM((SB,), jnp.int32),
                       pltpu.VMEM((SB,), jnp.int32),
                       pltpu.VMEM((SB, PACK), jnp.float32),
                       pltpu.VMEM((SB, PACK), jnp.float32),
                       pltpu.SemaphoreType.DMA])
    def k(t1_hbm, t2_hbm, row_hbm, col_hbm, g1_hbm, g2_hbm,
          idx1, idx2, buf1, buf2, sem):
        wid = lax.axis_index("s") * NC + lax.axis_index("c")
        base = wid * per
        end = base + per

        def step(bi, carry):
            start = jnp.minimum(base + bi * SB, end - SB)
            pltpu.sync_copy(row_hbm.at[pl.ds(start, SB)], idx1)
            pltpu.sync_copy(col_hbm.at[pl.ds(start, SB)], idx2)
            hs = []
            for j in range(SB // 128):
                sl = pl.ds(j * 128, 128)
                hs.append(pltpu.async_copy(t1_hbm.at[idx1.at[sl]], buf1.at[sl], sem))
                hs.append(pltpu.async_copy(t2_hbm.at[idx2.at[sl]], buf2.at[sl], sem))
            for hcopy in hs:
                hcopy.wait()
            pltpu.sync_copy(buf1, g1_hbm.at[pl.ds(start, SB)])
            pltpu.sync_copy(buf2, g2_hbm.at[pl.ds(start, SB)])
            return carry

        lax.fori_loop(0, nb, step, 0)

    return k(t1, t2, row, col)


def _sc_scatter(s_arr, row, n):
    e = row.shape[0]
    q = n // 4
    acc_rows = ((q + 1 + NS - 1) // NS) * NS
    share = acc_rows // NS
    per = e // NS
    SB = 256
    DB = 2
    nb = (per + SB * DB - 1) // (SB * DB)
    wshare = (q + NS - 1) // NS
    mesh = plsc.VectorSubcoreMesh(core_axis_name="c", subcore_axis_name="s",
                                  num_cores=NC, num_subcores=NS)

    @functools.partial(
        pl.kernel, mesh=mesh,
        compiler_params=pltpu.CompilerParams(use_tc_tiling_on_sc=False),
        out_type=jax.ShapeDtypeStruct((n, PACK), jnp.float32),
        scratch_types=[pltpu.VMEM((DB, SB), jnp.int32),
                       pltpu.VMEM((DB * (SB // 128), 128), jnp.int32),
                       pltpu.VMEM((DB * SB, PACK), jnp.float32),
                       pltpu.VMEM((128, PACK), jnp.float32),
                       pltpu.VMEM_SHARED((acc_rows, PACK), jnp.float32),
                       pltpu.SemaphoreType.DMA,
                       pltpu.SemaphoreType.DMA])
    def k(s_hbm, row_hbm, out_hbm, idxb, lidx, datab, zb, acc, seml, sema):
        ci = lax.axis_index("c")
        si = lax.axis_index("s")
        ebase = si * per
        eend = ebase + per

        def zrow(r, carry):
            for j in range(PACK // 16):
                zb[r, pl.ds(j * 16, 16)] = jnp.zeros((16,), jnp.float32)
            return carry
        lax.fori_loop(0, 128, zrow, 0)

        def phase(p, carry):
            nbase = (2 * p + ci) * q
            zbase = si * share

            def zcp(kk, c2):
                zs = jnp.minimum(zbase + kk * 128, zbase + share - 128)
                pltpu.sync_copy(zb, acc.at[pl.ds(zs, 128)])
                return c2
            lax.fori_loop(0, (share + 127) // 128, zcp, 0)
            plsc.subcore_barrier()

            def step(bi, c2):
                noms, starts, hl = [], [], []
                for j in range(DB):
                    nominal = ebase + (bi * DB + j) * SB
                    start = jnp.minimum(nominal, eend - SB)
                    noms.append(nominal)
                    starts.append(start)
                    hl.append(pltpu.async_copy(row_hbm.at[pl.ds(start, SB)],
                                               idxb.at[j], seml))
                    hl.append(pltpu.async_copy(
                        s_hbm.at[pl.ds(start, SB)],
                        datab.at[pl.ds(j * SB, SB)], seml))
                ha = []
                for j in range(DB):
                    hl[2 * j].wait()
                    hl[2 * j + 1].wait()
                    for g in range(SB // 16):
                        qq = g * 16
                        v = idxb[j, pl.ds(qq, 16)]
                        ev = starts[j] + qq + lax.iota(jnp.int32, 16)
                        ok = (v >= nbase) & (v < nbase + q) & (ev >= noms[j])
                        lidx[j * (SB // 128) + qq // 128,
                             pl.ds(qq % 128, 16)] = jnp.where(ok, v - nbase, q)
                    for m in range(SB // 128):
                        ha.append(pltpu.async_copy(
                            datab.at[pl.ds(j * SB + m * 128, 128)],
                            acc.at[lidx.at[j * (SB // 128) + m]], sema,
                            add=True))
                for hc in ha:
                    hc.wait()
                return c2

            lax.fori_loop(0, nb, step, 0)
            plsc.subcore_barrier()
            wstart = jnp.minimum(si * wshare, q - wshare)
            pltpu.sync_copy(acc.at[pl.ds(wstart, wshare)],
                            out_hbm.at[pl.ds(nbase + wstart, wshare)])
            return carry

        lax.fori_loop(0, 2, phase, 0)

    return k(s_arr, row)


def kernel(x, edge_index, edge_attr, x_coord, cond, params):
    n = x.shape[0]
    row = edge_index[0]
    col = edge_index[1]
    e = row.shape[0]
    ea2 = edge_attr[:, None]
    nl = len(params['layers'])
    cnl = len(params['cond']['gcl'])

    def run_layer(pp, t1, t2, normalize):
        g1, g2 = _sc_gather(t1, t2, row, col)
        s_arr = _edge_call(g1, g2, ea2, pp, normalize)
        return _sc_scatter(s_arr, row, n)

    cpp = [_prep(params['cond']['gcl'][l], 1) for l in range(cnl)]
    hc = _lin_call(cond, params['cond']['emb_in']['w'],
                   params['cond']['emb_in']['b'])
    coord = x_coord
    hf, t1, t2 = _node_call(hc, coord, proj=cpp[0])
    for l in range(cnl):
        agg = run_layer(cpp[l], t1, t2, normalize=False)
        if l < cnl - 1:
            hf, coord, t1, t2 = _node_call(hf, coord, upd=(agg, cpp[l]),
                                           proj=cpp[l + 1])
        else:
            hf, hsum = _node_call(hf, coord, upd=(agg, cpp[l]),
                                  emit_mean=True, emit_coord=False)
    g = hsum[0] / n
    g = params['cond']['emb_out']['w'] @ g + params['cond']['emb_out']['b']
    embed = params['cond']['fc']['w'] @ g + params['cond']['fc']['b']
    embed = embed.reshape(nl, 2, HID)
    scales, biases = embed[:, 0, :], embed[:, 1, :]

    we = params['edge_emb']['w'][:, 0]
    be = params['edge_emb']['b']
    mpp = [_prep(params['layers'][l], HID, we, be) for l in range(nl)]
    hv = _lin_call(x, params['node_emb']['w'], params['node_emb']['b'])
    coord = x_coord
    hf, t1, t2 = _node_call(hv, coord, film=(scales[0:1], biases[0:1]),
                            proj=mpp[0])
    for l in range(nl):
        agg = run_layer(mpp[l], t1, t2, normalize=True)
        if l < nl - 1:
            hf, coord, t1, t2 = _node_call(
                hf, coord, upd=(agg, mpp[l]),
                film=(scales[l + 1:l + 2], biases[l + 1:l + 2]),
                proj=mpp[l + 1])
        else:
            hf, coord, hsum = _node_call(hf, coord, upd=(agg, mpp[l]),
                                         emit_mean=True)
    ge = hsum[0] / n
    o = jnp.concatenate([ge, hf[n - 1]])
    o = jax.nn.relu(params['pred'][0]['w'] @ o + params['pred'][0]['b'])
    o = jax.nn.relu(params['pred'][1]['w'] @ o + params['pred'][1]['b'])
    o = params['pred'][2]['w'] @ o + params['pred'][2]['b']
    return o.reshape(16, 7), coord
